# baseline (speedup 1.0000x reference)
import functools

import jax
import jax.numpy as jnp
from jax import lax
from jax.experimental import pallas as pl
from jax.experimental.pallas import tpu as pltpu
from jax.experimental.pallas import tpu_sc as plsc

NC = 2
NS = 16
LANES = 16
N_OUT = 2048


@functools.lru_cache(maxsize=None)
def _build(B, EC, Dh):
    NBG = EC // LANES
    rpt = N_OUT // NS
    nvec = Dh // LANES

    mesh = plsc.VectorSubcoreMesh(
        core_axis_name="c", subcore_axis_name="s",
        num_cores=NC, num_subcores=NS)

    @functools.partial(
        pl.kernel,
        out_type=jax.ShapeDtypeStruct((B, N_OUT, NC, Dh), jnp.float32),
        mesh=mesh,
        scratch_types=[
            pltpu.VMEM((EC,), jnp.int32),
            pltpu.VMEM((EC,), jnp.int32),
            pltpu.VMEM((EC,), jnp.float32),
            pltpu.VMEM((EC + 2 * LANES,), jnp.int32),
            pltpu.VMEM((EC + 2 * LANES,), jnp.int32),
            pltpu.VMEM((EC + 2 * LANES,), jnp.float32),
            pltpu.VMEM((LANES, Dh), jnp.float32),
            pltpu.VMEM((rpt + 1, Dh), jnp.float32),
            pltpu.SemaphoreType.DMA,
        ],
        compiler_params=pltpu.CompilerParams(needs_layout_passes=False),
    )
    def run(score_h, idx_h, disp_h, w2_h, out_h,
            ia_v, da_v, sa_v, ci_v, cd_v, cs_v, gbuf, acc, sem):
        c = lax.axis_index("c")
        s = lax.axis_index("s")
        row0 = s * rpt
        zero = jnp.zeros((LANES,), jnp.float32)

        for b in range(B):
            pltpu.sync_copy(idx_h.at[b], ia_v)
            pltpu.sync_copy(disp_h.at[b], da_v)
            pltpu.sync_copy(score_h.at[b], sa_v)

            def zrow(r, carry):
                for v in range(nvec):
                    acc[r, pl.ds(v * LANES, LANES)] = zero
                return carry

            lax.fori_loop(0, rpt + 1, zrow, 0)

            def scan(g, cur):
                sl = pl.ds(g * LANES, LANES)
                lr = da_v[sl] - row0
                m = (lr >= 0) & (lr < rpt)
                plsc.store_compressed(cd_v.at[pl.ds(cur, LANES)], lr, mask=m)
                plsc.store_compressed(ci_v.at[pl.ds(cur, LANES)], ia_v[sl], mask=m)
                plsc.store_compressed(cs_v.at[pl.ds(cur, LANES)], sa_v[sl], mask=m)
                return cur + plsc.all_reduce_population_count(m)[0]

            m_cnt = lax.fori_loop(0, NBG, scan, jnp.int32(0))

            pad = pl.ds(m_cnt, LANES)
            cd_v[pad] = jnp.full((LANES,), rpt, jnp.int32)
            ci_v[pad] = jnp.zeros((LANES,), jnp.int32)
            cs_v[pad] = zero

            def group(gg, carry):
                k0 = gg * LANES
                iv2 = ci_v[pl.ds(k0, LANES)] * 2 + c
                pltpu.async_copy(w2_h.at[iv2], gbuf, sem).wait()

                def row(r, carry2):
                    lr = cd_v[pl.ds(k0 + r, LANES)][0]
                    sv = jnp.broadcast_to(cs_v[pl.ds(k0 + r, LANES)][0],
                                          (LANES,))
                    for v in range(nvec):
                        sl = pl.ds(v * LANES, LANES)
                        acc[lr, sl] = acc[lr, sl] + gbuf[r, sl] * sv
                    return carry2

                lax.fori_loop(0, LANES, row, 0)
                return carry

            ng = (m_cnt + LANES - 1) // LANES
            lax.fori_loop(0, ng, group, 0)
            pltpu.sync_copy(acc.at[pl.ds(0, rpt)],
                            out_h.at[b, pl.ds(row0, rpt), c])

    return run


def kernel(score, indices, dispatch, n, weight):
    B, E, C = score.shape
    V, D = weight.shape
    Dh = D // NC
    EC = E * C
    w2 = weight.reshape(V * NC, Dh)
    score2 = score.reshape(B, EC)
    idx2 = indices.reshape(B, EC)
    disp2 = jnp.minimum(dispatch.reshape(B, EC), n - 1).astype(jnp.int32)
    out4 = _build(B, EC, Dh)(score2, idx2, disp2, w2)
    return out4.reshape(B, N_OUT, D)


# speedup vs baseline: 1.1617x; 1.1617x over previous
import functools

import jax
import jax.numpy as jnp
from jax import lax
from jax.experimental import pallas as pl
from jax.experimental.pallas import tpu as pltpu
from jax.experimental.pallas import tpu_sc as plsc

NC = 2
NS = 16
LANES = 16
N_OUT = 2048


@functools.lru_cache(maxsize=None)
def _build(B, EC, Dh):
    NBG = EC // LANES
    rpt = N_OUT // NS
    nvec = Dh // LANES

    mesh = plsc.VectorSubcoreMesh(
        core_axis_name="c", subcore_axis_name="s",
        num_cores=NC, num_subcores=NS)

    @functools.partial(
        pl.kernel,
        out_type=jax.ShapeDtypeStruct((B, N_OUT, NC, Dh), jnp.float32),
        mesh=mesh,
        scratch_types=[
            pltpu.VMEM((EC,), jnp.int32),
            pltpu.VMEM((EC,), jnp.int32),
            pltpu.VMEM((EC,), jnp.float32),
            pltpu.VMEM((EC + 2 * LANES,), jnp.int32),
            pltpu.VMEM((EC + 2 * LANES,), jnp.int32),
            pltpu.VMEM((EC + 2 * LANES,), jnp.float32),
            pltpu.VMEM((LANES, Dh), jnp.float32),
            pltpu.VMEM((LANES, Dh), jnp.float32),
            pltpu.VMEM((rpt + 1, Dh), jnp.float32),
            pltpu.SemaphoreType.DMA,
            pltpu.SemaphoreType.DMA,
            pltpu.SemaphoreType.DMA,
        ],
        compiler_params=pltpu.CompilerParams(needs_layout_passes=False),
    )
    def run(score_h, idx_h, disp_h, w2_h, out_h,
            ia_v, da_v, sa_v, ci_v, cd_v, cs_v, gbuf0, gbuf1, acc,
            sem_in, sem0, sem1):
        c = lax.axis_index("c")
        s = lax.axis_index("s")
        row0 = s * rpt
        zero = jnp.zeros((LANES,), jnp.float32)
        gbufs = (gbuf0, gbuf1)
        sems = (sem0, sem1)

        def fetch(gg, buf, sem):
            iv2 = ci_v[pl.ds(gg * LANES, LANES)] * 2 + c
            pltpu.async_copy(w2_h.at[iv2], buf, sem)

        def drain(buf, sem):
            pltpu.make_async_copy(w2_h.at[pl.ds(0, LANES)], buf, sem).wait()

        def accumulate(k0, buf):
            def row(r, carry2):
                lr = cd_v[pl.ds(k0 + r, LANES)][0]
                sv = jnp.broadcast_to(cs_v[pl.ds(k0 + r, LANES)][0], (LANES,))
                for v in range(nvec):
                    sl = pl.ds(v * LANES, LANES)
                    plsc.addupdate(acc.at[lr, sl], buf[r, sl] * sv)
                return carry2
            lax.fori_loop(0, LANES, row, 0)

        for b in range(B):
            cp_i = pltpu.async_copy(idx_h.at[b], ia_v, sem_in)
            cp_d = pltpu.async_copy(disp_h.at[b], da_v, sem_in)
            cp_s = pltpu.async_copy(score_h.at[b], sa_v, sem_in)

            def zrow(r, carry):
                for v in range(nvec):
                    acc[r, pl.ds(v * LANES, LANES)] = zero
                return carry

            lax.fori_loop(0, rpt + 1, zrow, 0)
            cp_i.wait()
            cp_d.wait()
            cp_s.wait()

            def scan(g, cur):
                sl = pl.ds(g * LANES, LANES)
                lr = da_v[sl] - row0
                m = (lr >= 0) & (lr < rpt)
                plsc.store_compressed(cd_v.at[pl.ds(cur, LANES)], lr, mask=m)
                plsc.store_compressed(ci_v.at[pl.ds(cur, LANES)], ia_v[sl],
                                      mask=m)
                plsc.store_compressed(cs_v.at[pl.ds(cur, LANES)], sa_v[sl],
                                      mask=m)
                return cur + plsc.all_reduce_population_count(m)[0]

            m_cnt = lax.fori_loop(0, NBG, scan, jnp.int32(0))

            pad = pl.ds(m_cnt, LANES)
            cd_v[pad] = jnp.full((LANES,), rpt, jnp.int32)
            ci_v[pad] = jnp.zeros((LANES,), jnp.int32)
            cs_v[pad] = zero

            ng = (m_cnt + LANES - 1) // LANES
            pl.when(ng > 0)(lambda: fetch(0, gbuf0, sem0))

            def group(gg, carry):
                for p in range(2):
                    @pl.when(lax.rem(gg, 2) == p)
                    def _():
                        pl.when(gg + 1 < ng)(
                            lambda: fetch(gg + 1, gbufs[1 - p], sems[1 - p]))
                        drain(gbufs[p], sems[p])
                        accumulate(gg * LANES, gbufs[p])
                return carry

            lax.fori_loop(0, ng, group, 0)
            pltpu.sync_copy(acc.at[pl.ds(0, rpt)],
                            out_h.at[b, pl.ds(row0, rpt), c])

    return run


def kernel(score, indices, dispatch, n, weight):
    B, E, C = score.shape
    V, D = weight.shape
    Dh = D // NC
    EC = E * C
    w2 = weight.reshape(V * NC, Dh)
    score2 = score.reshape(B, EC)
    idx2 = indices.reshape(B, EC)
    disp2 = jnp.minimum(dispatch.reshape(B, EC), n - 1).astype(jnp.int32)
    out4 = _build(B, EC, Dh)(score2, idx2, disp2, w2)
    return out4.reshape(B, N_OUT, D)


# speedup vs baseline: 3.2349x; 2.7845x over previous
import functools

import jax
import jax.numpy as jnp
from jax import lax
from jax.experimental import pallas as pl
from jax.experimental.pallas import tpu as pltpu
from jax.experimental.pallas import tpu_sc as plsc

NC = 2
NS = 16
LANES = 16
N_OUT = 2048


@functools.lru_cache(maxsize=None)
def _build(B, EC, D):
    NBG = EC // LANES
    rpt = N_OUT // (NC * NS)
    nvec = D // LANES

    mesh = plsc.VectorSubcoreMesh(
        core_axis_name="c", subcore_axis_name="s",
        num_cores=NC, num_subcores=NS)

    @functools.partial(
        pl.kernel,
        out_type=jax.ShapeDtypeStruct((B, N_OUT, D), jnp.float32),
        mesh=mesh,
        scratch_types=[
            pltpu.VMEM((EC,), jnp.int32),
            pltpu.VMEM((EC,), jnp.int32),
            pltpu.VMEM((EC,), jnp.float32),
            pltpu.VMEM((EC + 2 * LANES,), jnp.int32),
            pltpu.VMEM((EC + 2 * LANES,), jnp.int32),
            pltpu.VMEM((EC + 2 * LANES,), jnp.float32),
            pltpu.VMEM((LANES, D), jnp.float32),
            pltpu.VMEM((LANES, D), jnp.float32),
            pltpu.VMEM((rpt, D), jnp.float32),
            pltpu.SemaphoreType.DMA,
            pltpu.SemaphoreType.DMA,
            pltpu.SemaphoreType.DMA,
        ],
        compiler_params=pltpu.CompilerParams(needs_layout_passes=False),
    )
    def run(score_h, idx_h, disp_h, w_h, out_h,
            ia_v, da_v, sa_v, ci_v, cd_v, cs_v, gbuf0, gbuf1, acc,
            sem_in, sem0, sem1):
        c = lax.axis_index("c")
        s = lax.axis_index("s")
        row0 = (c * NS + s) * rpt
        zero = jnp.zeros((LANES,), jnp.float32)
        gbufs = (gbuf0, gbuf1)
        sems = (sem0, sem1)

        def fetch(gg, buf, sem):
            iv = ci_v[pl.ds(gg * LANES, LANES)]
            pltpu.async_copy(w_h.at[iv], buf, sem)

        def drain(buf, sem):
            pltpu.make_async_copy(w_h.at[pl.ds(0, LANES)], buf, sem).wait()

        def accumulate(k0, buf):
            def row(r, carry2):
                lr = cd_v[pl.ds(k0 + r, LANES)][0]
                sv = jnp.broadcast_to(cs_v[pl.ds(k0 + r, LANES)][0], (LANES,))
                for v in range(nvec):
                    sl = pl.ds(v * LANES, LANES)
                    plsc.addupdate(acc.at[lr, sl], buf[r, sl] * sv)
                return carry2
            lax.fori_loop(0, LANES, row, 0)

        for b in range(B):
            cp_i = pltpu.async_copy(idx_h.at[b], ia_v, sem_in)
            cp_d = pltpu.async_copy(disp_h.at[b], da_v, sem_in)
            cp_s = pltpu.async_copy(score_h.at[b], sa_v, sem_in)

            def zrow(r, carry):
                for v in range(nvec):
                    acc[r, pl.ds(v * LANES, LANES)] = zero
                return carry

            lax.fori_loop(0, rpt, zrow, 0)
            cp_i.wait()
            cp_d.wait()
            cp_s.wait()

            def scan(g, cur):
                sl = pl.ds(g * LANES, LANES)
                lr = da_v[sl] - row0
                m = (lr >= 0) & (lr < rpt)
                plsc.store_compressed(cd_v.at[pl.ds(cur, LANES)], lr, mask=m)
                plsc.store_compressed(ci_v.at[pl.ds(cur, LANES)], ia_v[sl],
                                      mask=m)
                plsc.store_compressed(cs_v.at[pl.ds(cur, LANES)], sa_v[sl],
                                      mask=m)
                return cur + plsc.all_reduce_population_count(m)[0]

            m_cnt = lax.fori_loop(0, NBG, scan, jnp.int32(0))

            pad = pl.ds(m_cnt, LANES)
            cd_v[pad] = jnp.full((LANES,), rpt - 1, jnp.int32)
            ci_v[pad] = jnp.zeros((LANES,), jnp.int32)
            cs_v[pad] = zero

            ng = (m_cnt + LANES - 1) // LANES
            pl.when(ng > 0)(lambda: fetch(0, gbuf0, sem0))

            def group(gg, carry):
                for p in range(2):
                    @pl.when(lax.rem(gg, 2) == p)
                    def _():
                        pl.when(gg + 1 < ng)(
                            lambda: fetch(gg + 1, gbufs[1 - p], sems[1 - p]))
                        drain(gbufs[p], sems[p])
                        accumulate(gg * LANES, gbufs[p])
                return carry

            lax.fori_loop(0, ng, group, 0)
            pltpu.sync_copy(acc.at[pl.ds(0, rpt)],
                            out_h.at[b, pl.ds(row0, rpt)])

    return run


def kernel(score, indices, dispatch, n, weight):
    B, E, C = score.shape
    V, D = weight.shape
    EC = E * C
    score2 = score.reshape(B, EC)
    idx2 = indices.reshape(B, EC)
    disp2 = jnp.minimum(dispatch.reshape(B, EC), n - 1).astype(jnp.int32)
    return _build(B, EC, D)(score2, idx2, disp2, weight)
